# baseline (speedup 1.0000x reference)
import functools

import jax
import jax.numpy as jnp
from jax import lax
from jax.experimental import pallas as pl
from jax.experimental.pallas import tpu as pltpu
from jax.experimental.pallas import tpu_sc as plsc

_ALL_ANCHORS = [(12, 16), (19, 36), (40, 28), (36, 75), (76, 55),
                (72, 146), (142, 110), (192, 243), (459, 401)]
_ANCHOR_MASK = [6, 7, 8]
_AW = [float(_ALL_ANCHORS[i][0]) for i in _ANCHOR_MASK]
_AH = [float(_ALL_ANCHORS[i][1]) for i in _ANCHOR_MASK]
_N_ATTRS = 85
_N_ANCHORS = 3
_G = 19
_CH = 255
_HSPAN = 11
_ROWS_DMA = 632
_ROW_OFF = 456
_C0S = (16, 32, 48, 64, 69)


def _sc_body(x_hbm, stride_hbm, out_hbm, in_v, out_v, sv_v, sem):
    del sem
    cid = lax.axis_index("c")
    sid = lax.axis_index("s")
    wid = sid * 2 + cid
    b = wid // 2
    half = wid % 2
    h0 = half * 8

    pltpu.sync_copy(stride_hbm, sv_v)
    sv = sv_v[...]

    iota = lax.broadcasted_iota(jnp.int32, (16,), 0)
    is_wh = (iota == 2) | (iota == 3)
    sgn0 = jnp.where(is_wh, jnp.float32(1.0), jnp.float32(-1.0))
    mul0 = []
    for a in range(_N_ANCHORS):
        mul0.append(jnp.where(iota < 2, sv,
                    jnp.where(iota == 2, _AW[a],
                    jnp.where(iota == 3, _AH[a], 1.0))))

    def do_cell(r0, w, gy_f):
        gx = w.astype(jnp.float32)
        widx = jnp.full((16,), w, jnp.int32)
        addv = jnp.where(iota == 0, gx, jnp.where(iota == 1, gy_f, 0.0))
        for a in range(_N_ANCHORS):
            r = r0 + _N_ANCHORS * w + a
            base = a * _N_ATTRS
            t = plsc.load_gather(in_v, [base + iota, widx])
            p = jnp.exp(t * sgn0)
            val = jnp.where(is_wh, p, 1.0 / (1.0 + p))
            out_v[r, pl.ds(0, 16)] = (val + addv) * mul0[a]
            for c0 in _C0S:
                t2 = plsc.load_gather(in_v, [base + c0 + iota, widx])
                out_v[r, pl.ds(c0, 16)] = 1.0 / (1.0 + jnp.exp(-t2))

    def hbody(hl, carry):
        pltpu.sync_copy(x_hbm.at[b, :, h0 + hl, :], in_v)
        gy = (h0 + hl).astype(jnp.float32)

        def cell(w, carry2):
            do_cell(_N_ANCHORS * _G * hl, w, gy)
            return carry2

        lax.fori_loop(0, _G, cell, 0)
        return carry

    lax.fori_loop(0, _HSPAN, hbody, 0)

    hx = jnp.minimum(h0 + _HSPAN, _G - 1)
    pltpu.sync_copy(x_hbm.at[b, :, hx, :], in_v)
    gyx = hx.astype(jnp.float32)

    def xcell(w, carry2):
        do_cell(_N_ANCHORS * _G * _HSPAN, w, gyx)
        return carry2

    lax.fori_loop(0, 2, xcell, 0)

    pltpu.sync_copy(out_v.at[pl.ds(0, _ROWS_DMA), :],
                    out_hbm.at[b, pl.ds(half * _ROW_OFF, _ROWS_DMA), :])


def kernel(x, input_dim):
    b, ch, h, w = x.shape
    hw = h * w
    stride = jnp.floor(jnp.asarray(input_dim, jnp.float32) / jnp.float32(h))
    stridevec = jnp.full((16,), stride, jnp.float32)

    mesh = plsc.VectorSubcoreMesh(core_axis_name="c", subcore_axis_name="s",
                                  num_cores=2, num_subcores=16)
    run = functools.partial(
        pl.kernel,
        out_type=jax.ShapeDtypeStruct((b, hw * _N_ANCHORS, _N_ATTRS),
                                      jnp.float32),
        mesh=mesh,
        compiler_params=pltpu.CompilerParams(needs_layout_passes=False,
                                             disable_bounds_checks=True),
        scratch_types=[
            pltpu.VMEM((_CH, _G), jnp.float32),
            pltpu.VMEM((_N_ANCHORS * _G * _HSPAN + 9, _N_ATTRS), jnp.float32),
            pltpu.VMEM((16,), jnp.float32),
            pltpu.SemaphoreType.DMA,
        ],
    )(_sc_body)
    return run(x, stridevec)


# speedup vs baseline: 7.2123x; 7.2123x over previous
import jax
import jax.numpy as jnp
from jax.experimental import pallas as pl

_ALL_ANCHORS = [(12, 16), (19, 36), (40, 28), (36, 75), (76, 55),
                (72, 146), (142, 110), (192, 243), (459, 401)]
_ANCHOR_MASK = [6, 7, 8]
_N_ATTRS = 85
_N_ANCHORS = 3
_BLK = 4
_LOG2E = 1.4426950408889634


def _yolo_body(stride_ref, x_ref, o_ref):
    stride = stride_ref[0, 0]
    shape = x_ref.shape[1:]

    k = jax.lax.broadcasted_iota(jnp.int32, shape, 0)
    col = jax.lax.broadcasted_iota(jnp.int32, shape, 1)
    c = k % _N_ATTRS
    is_wh = (c == 2) | (c == 3)
    sgn = jnp.where(is_wh, jnp.float32(_LOG2E), jnp.float32(-_LOG2E))
    gx = (col % 19).astype(jnp.float32)
    gy = (col // 19).astype(jnp.float32)
    add = jnp.where(c == 0, gx, jnp.where(c == 1, gy, 0.0))
    aw_tab = [float(_ALL_ANCHORS[i][0]) for i in _ANCHOR_MASK]
    ah_tab = [float(_ALL_ANCHORS[i][1]) for i in _ANCHOR_MASK]
    aw = jnp.where(k < _N_ATTRS, aw_tab[0],
                   jnp.where(k < 2 * _N_ATTRS, aw_tab[1], aw_tab[2]))
    ah = jnp.where(k < _N_ATTRS, ah_tab[0],
                   jnp.where(k < 2 * _N_ATTRS, ah_tab[1], ah_tab[2]))
    mul = jnp.where(c < 2, stride,
                    jnp.where(c == 2, aw, jnp.where(c == 3, ah, 1.0)))

    for img in range(_BLK):
        v = x_ref[img]
        p = jnp.exp2(v * sgn)
        r = 1.0 / (1.0 + p)
        out = (jnp.where(is_wh, p, r) + add) * mul
        t = out.T
        for a in range(_N_ANCHORS):
            o_ref[img, a::_N_ANCHORS, :] = t[:, a * _N_ATTRS:(a + 1) * _N_ATTRS]


def kernel(x, input_dim):
    b, ch, h, w = x.shape
    hw = h * w
    xr = x.reshape(b, ch, hw)
    stride = jnp.floor(jnp.asarray(input_dim, jnp.float32) / jnp.float32(h))
    stride = stride.reshape(1, 1)

    out = pl.pallas_call(
        _yolo_body,
        grid=(b // _BLK,),
        in_specs=[
            pl.BlockSpec((1, 1), lambda i: (0, 0)),
            pl.BlockSpec((_BLK, ch, hw), lambda i: (i, 0, 0)),
        ],
        out_specs=pl.BlockSpec((_BLK, hw * _N_ANCHORS, _N_ATTRS),
                               lambda i: (i, 0, 0)),
        out_shape=jax.ShapeDtypeStruct((b, hw * _N_ANCHORS, _N_ATTRS),
                                       jnp.float32),
    )(stride, xr)
    return out
